# Initial kernel scaffold; baseline (speedup 1.0000x reference)
import functools

import jax
import jax.numpy as jnp
from jax import lax
from jax.experimental import pallas as pl
from jax.experimental.pallas import tpu as pltpu
from jax.experimental.pallas import tpu_sc as plsc

CH = 128


@functools.lru_cache(maxsize=None)
def _make_gather(num_rows, dim, num_chunks):
    mesh = plsc.VectorSubcoreMesh(core_axis_name="c", subcore_axis_name="s")
    nc, ns = mesh.num_cores, mesh.num_subcores
    nw = nc * ns
    assert num_chunks % nw == 0
    chunks_per_w = num_chunks // nw

    @functools.partial(
        pl.kernel,
        out_type=jax.ShapeDtypeStruct((num_rows, dim), jnp.float32),
        mesh=mesh,
        scratch_types=[
            pltpu.VMEM((chunks_per_w, CH), jnp.int32),
            pltpu.VMEM((CH, dim), jnp.float32),
            pltpu.SemaphoreType.DMA,
        ],
    )
    def gather_kernel(idx_hbm, table_hbm, out_hbm, idx_v, rows_v, sem):
        wid = lax.axis_index("s") * nc + lax.axis_index("c")
        base_chunk = wid * chunks_per_w
        pltpu.sync_copy(idx_hbm.at[pl.ds(base_chunk, chunks_per_w)], idx_v)

        @pl.loop(0, chunks_per_w)
        def _(j):
            pltpu.async_copy(table_hbm.at[idx_v.at[j]], rows_v, sem).wait()
            pltpu.sync_copy(
                rows_v, out_hbm.at[pl.ds((base_chunk + j) * CH, CH)]
            )

    return gather_kernel


def kernel(x, weight):
    b, h = x.shape
    n, d = weight.shape
    idx = x.reshape(-1).astype(jnp.int32)
    num_rows = idx.shape[0]
    assert num_rows % CH == 0
    idx2 = idx.reshape(num_rows // CH, CH)
    out = _make_gather(num_rows, d, num_rows // CH)(idx2, weight)
    return out.reshape(b, h, d)


# speedup vs baseline: 1.0222x; 1.0222x over previous
import functools

import jax
import jax.numpy as jnp
from jax import lax
from jax.experimental import pallas as pl
from jax.experimental.pallas import tpu as pltpu
from jax.experimental.pallas import tpu_sc as plsc

CH = 128


@functools.lru_cache(maxsize=None)
def _make_gather(num_rows, dim, num_chunks):
    mesh = plsc.VectorSubcoreMesh(core_axis_name="c", subcore_axis_name="s")
    nc, ns = mesh.num_cores, mesh.num_subcores
    nw = nc * ns
    assert num_chunks % nw == 0
    chunks_per_w = num_chunks // nw

    @functools.partial(
        pl.kernel,
        out_type=jax.ShapeDtypeStruct((num_rows, dim), jnp.float32),
        mesh=mesh,
        scratch_types=[
            pltpu.VMEM((chunks_per_w, CH), jnp.int32),
            pltpu.VMEM((CH, dim), jnp.float32),
            pltpu.SemaphoreType.DMA,
        ],
        compiler_params=pltpu.CompilerParams(use_tc_tiling_on_sc=False),
    )
    def gather_kernel(idx_hbm, table_hbm, out_hbm, idx_v, rows_v, sem):
        wid = lax.axis_index("s") * nc + lax.axis_index("c")
        base_chunk = wid * chunks_per_w
        pltpu.sync_copy(idx_hbm.at[pl.ds(base_chunk, chunks_per_w)], idx_v)

        @pl.loop(0, chunks_per_w)
        def _(j):
            pltpu.async_copy(table_hbm.at[idx_v.at[j]], rows_v, sem).wait()
            pltpu.sync_copy(
                rows_v, out_hbm.at[pl.ds((base_chunk + j) * CH, CH)]
            )

    return gather_kernel


def kernel(x, weight):
    b, h = x.shape
    n, d = weight.shape
    idx = x.reshape(-1).astype(jnp.int32)
    num_rows = idx.shape[0]
    assert num_rows % CH == 0
    idx2 = idx.reshape(num_rows // CH, CH)
    out = _make_gather(num_rows, d, num_rows // CH)(idx2, weight)
    return out.reshape(b, h, d)


# speedup vs baseline: 1.1106x; 1.0865x over previous
import functools

import jax
import jax.numpy as jnp
from jax import lax
from jax.experimental import pallas as pl
from jax.experimental.pallas import tpu as pltpu
from jax.experimental.pallas import tpu_sc as plsc

CH = 128


R = 8
G = 4


@functools.lru_cache(maxsize=None)
def _make_gather(num_rows, dim, num_chunks):
    mesh = plsc.VectorSubcoreMesh(core_axis_name="c", subcore_axis_name="s")
    nc, ns = mesh.num_cores, mesh.num_subcores
    nw = nc * ns
    assert num_chunks % nw == 0
    chunks_per_w = num_chunks // nw
    steady = chunks_per_w - 2 * G
    assert steady % R == 0 and chunks_per_w > 2 * R

    @functools.partial(
        pl.kernel,
        out_type=jax.ShapeDtypeStruct((num_rows, dim), jnp.float32),
        mesh=mesh,
        scratch_types=[
            pltpu.VMEM((chunks_per_w, CH), jnp.int32),
            pltpu.VMEM((R, CH, dim), jnp.float32),
            pltpu.SemaphoreType.DMA((R,)),
            pltpu.SemaphoreType.DMA((R,)),
        ],
        compiler_params=pltpu.CompilerParams(use_tc_tiling_on_sc=False),
    )
    def gather_kernel(idx_hbm, table_hbm, out_hbm, idx_v, rows_v, gsem, ssem):
        wid = lax.axis_index("s") * nc + lax.axis_index("c")
        base_chunk = wid * chunks_per_w
        pltpu.sync_copy(idx_hbm.at[pl.ds(base_chunk, chunks_per_w)], idx_v)

        def fire_gather(j, rr):
            pltpu.async_copy(
                table_hbm.at[idx_v.at[j]], rows_v.at[rr], gsem.at[rr]
            )

        def wait_gather(rr):
            pltpu.make_async_copy(
                out_hbm.at[pl.ds(0, CH)], rows_v.at[rr], gsem.at[rr]
            ).wait()

        def fire_scatter(j, rr):
            pltpu.async_copy(
                rows_v.at[rr],
                out_hbm.at[pl.ds((base_chunk + j) * CH, CH)],
                ssem.at[rr],
            )

        def wait_scatter(rr):
            pltpu.make_async_copy(
                rows_v.at[rr], out_hbm.at[pl.ds(0, CH)], ssem.at[rr]
            ).wait()

        for j in range(R - G):
            fire_gather(j, j % R)
        for i in range(G):
            fire_gather(i + (R - G), (i + (R - G)) % R)
            wait_gather(i % R)
            fire_scatter(i, i % R)

        @pl.loop(0, steady // R)
        def _(o):
            i0 = G + o * R
            for k in range(R):
                i = i0 + k
                bpre = (G + k + (R - G)) % R
                wait_scatter(bpre)
                fire_gather(i + (R - G), bpre)
                b = (G + k) % R
                wait_gather(b)
                fire_scatter(i, b)

        for i in range(chunks_per_w - G, chunks_per_w):
            b = i % R
            wait_gather(b)
            fire_scatter(i, b)
        for rr in range(R):
            wait_scatter(rr)

    return gather_kernel


def kernel(x, weight):
    b, h = x.shape
    n, d = weight.shape
    idx = x.reshape(-1).astype(jnp.int32)
    num_rows = idx.shape[0]
    assert num_rows % CH == 0
    idx2 = idx.reshape(num_rows // CH, CH)
    out = _make_gather(num_rows, d, num_rows // CH)(idx2, weight)
    return out.reshape(b, h, d)


# speedup vs baseline: 1.9386x; 1.7455x over previous
import functools

import jax
import jax.numpy as jnp
from jax import lax
from jax.experimental import pallas as pl
from jax.experimental.pallas import tpu as pltpu
from jax.experimental.pallas import tpu_sc as plsc

CH = 128


R = 8
G = 4


@functools.lru_cache(maxsize=None)
def _make_gather(num_rows, dim, num_chunks):
    mesh = plsc.VectorSubcoreMesh(core_axis_name="c", subcore_axis_name="s")
    nc, ns = mesh.num_cores, mesh.num_subcores
    nw = nc * ns
    assert num_chunks % nw == 0
    chunks_per_w = num_chunks // nw
    steady = chunks_per_w - 2 * G
    assert steady % R == 0 and chunks_per_w > 2 * R

    @functools.partial(
        pl.kernel,
        out_type=jax.ShapeDtypeStruct((num_rows, dim), jnp.float32),
        mesh=mesh,
        scratch_types=[
            pltpu.VMEM((chunks_per_w, CH), jnp.int32),
            pltpu.VMEM((R, CH, dim), jnp.float32),
            pltpu.SemaphoreType.DMA((R,)),
            pltpu.SemaphoreType.DMA((R,)),
        ],
        compiler_params=pltpu.CompilerParams(use_tc_tiling_on_sc=False),
    )
    def gather_kernel(idx_hbm, table_hbm, out_hbm, idx_v, rows_v, gsem, ssem):
        wid = lax.axis_index("s") * nc + lax.axis_index("c")
        base_chunk = wid * chunks_per_w
        pltpu.sync_copy(idx_hbm.at[pl.ds(base_chunk, chunks_per_w)], idx_v)

        def fire_gather(j, rr):
            pltpu.async_copy(
                table_hbm.at[idx_v.at[j]], rows_v.at[rr], gsem.at[rr]
            )

        def wait_gather(rr):
            pltpu.make_async_copy(
                out_hbm.at[pl.ds(0, CH)], rows_v.at[rr], gsem.at[rr]
            ).wait()

        def fire_scatter(j, rr):
            pltpu.async_copy(
                rows_v.at[rr],
                out_hbm.at[pl.ds((base_chunk + j) * CH, CH)],
                ssem.at[rr],
            )

        def wait_scatter(rr):
            pltpu.make_async_copy(
                rows_v.at[rr], out_hbm.at[pl.ds(0, CH)], ssem.at[rr]
            ).wait()

        for j in range(R - G):
            fire_gather(j, j % R)
        for i in range(G):
            fire_gather(i + (R - G), (i + (R - G)) % R)
            wait_gather(i % R)
            fire_scatter(i, i % R)

        @pl.loop(0, steady // R)
        def _(o):
            i0 = G + o * R
            for k in range(R):
                i = i0 + k
                bpre = (G + k + (R - G)) % R
                wait_scatter(bpre)
                fire_gather(i + (R - G), bpre)
                b = (G + k) % R
                wait_gather(b)
                fire_scatter(i, b)

        for i in range(chunks_per_w - G, chunks_per_w):
            b = i % R
            wait_gather(b)
            fire_scatter(i, b)
        for rr in range(R):
            wait_scatter(rr)

    return gather_kernel


def kernel(x, weight):
    b, h = x.shape
    n, d = weight.shape
    idx = jnp.swapaxes(x, 0, 1).reshape(-1).astype(jnp.int32)
    num_rows = idx.shape[0]
    assert num_rows % CH == 0
    idx2 = idx.reshape(num_rows // CH, CH)
    out = _make_gather(num_rows, d, num_rows // CH)(idx2, weight)
    return out.reshape(h, b, d).transpose(1, 0, 2)


# speedup vs baseline: 1.9402x; 1.0008x over previous
import functools

import jax
import jax.numpy as jnp
from jax import lax
from jax.experimental import pallas as pl
from jax.experimental.pallas import tpu as pltpu
from jax.experimental.pallas import tpu_sc as plsc

CH = 128
R = 8
G = 4


@functools.lru_cache(maxsize=None)
def _make_gather(hist, batch, dim):
    mesh = plsc.VectorSubcoreMesh(core_axis_name="c", subcore_axis_name="s")
    nc, ns = mesh.num_cores, mesh.num_subcores
    nw = nc * ns
    assert batch % (nw * CH) == 0
    bw = batch // nw
    cph = bw // CH
    chunks_per_w = hist * cph
    steady = chunks_per_w - 2 * G
    assert steady % R == 0 and chunks_per_w > 2 * R

    @functools.partial(
        pl.kernel,
        out_type=jax.ShapeDtypeStruct((hist, batch, dim), jnp.float32),
        mesh=mesh,
        scratch_types=[
            pltpu.VMEM((hist, bw), jnp.int32),
            pltpu.VMEM((R, CH, dim), jnp.float32),
            pltpu.SemaphoreType.DMA((R,)),
            pltpu.SemaphoreType.DMA((R,)),
        ],
        compiler_params=pltpu.CompilerParams(use_tc_tiling_on_sc=False),
    )
    def gather_kernel(idx_hbm, table_hbm, out_hbm, idx_v, rows_v, gsem, ssem):
        wid = lax.axis_index("s") * nc + lax.axis_index("c")
        col0 = wid * bw
        pltpu.sync_copy(idx_hbm.at[:, pl.ds(col0, bw)], idx_v)

        def fire_gather(t, rr):
            h = t // cph
            c = t % cph
            pltpu.async_copy(
                table_hbm.at[idx_v.at[h, pl.ds(c * CH, CH)]],
                rows_v.at[rr],
                gsem.at[rr],
            )

        def wait_gather(rr):
            pltpu.make_async_copy(
                out_hbm.at[0, pl.ds(0, CH)], rows_v.at[rr], gsem.at[rr]
            ).wait()

        def fire_scatter(t, rr):
            h = t // cph
            c = t % cph
            pltpu.async_copy(
                rows_v.at[rr],
                out_hbm.at[h, pl.ds(col0 + c * CH, CH)],
                ssem.at[rr],
            )

        def wait_scatter(rr):
            pltpu.make_async_copy(
                rows_v.at[rr], out_hbm.at[0, pl.ds(0, CH)], ssem.at[rr]
            ).wait()

        for t in range(R - G):
            fire_gather(t, t % R)
        for i in range(G):
            fire_gather(i + (R - G), (i + (R - G)) % R)
            wait_gather(i % R)
            fire_scatter(i, i % R)

        @pl.loop(0, steady // R)
        def _(o):
            t0 = G + o * R
            for k in range(R):
                t = t0 + k
                bpre = (G + k + (R - G)) % R
                wait_scatter(bpre)
                fire_gather(t + (R - G), bpre)
                b = (G + k) % R
                wait_gather(b)
                fire_scatter(t, b)

        for t in range(chunks_per_w - G, chunks_per_w):
            b = t % R
            wait_gather(b)
            fire_scatter(t, b)
        for rr in range(R):
            wait_scatter(rr)

    return gather_kernel


def kernel(x, weight):
    b, h = x.shape
    n, d = weight.shape
    idx = jnp.swapaxes(x, 0, 1).astype(jnp.int32)
    out = _make_gather(h, b, d)(idx, weight)
    return out.transpose(1, 0, 2)
